# Initial kernel scaffold; baseline (speedup 1.0000x reference)
import jax
import jax.numpy as jnp
from jax.experimental import pallas as pl


def kernel(x, spin, params):
    raise NotImplementedError("write your pallas kernel here")


# speedup vs baseline: 2.3584x; 2.3584x over previous
import numpy as np
import jax
import jax.numpy as jnp
from jax.experimental import pallas as pl
from jax.experimental.pallas import tpu as pltpu

N_PART = 16
DIM = 3
H = 32
N_STEPS = 2
B_BLK = 128
W_E = N_PART * H

_EDGE_I = np.asarray([i for i in range(N_PART) for j in range(N_PART) if i != j])
_EDGE_J = np.asarray([j for i in range(N_PART) for j in range(N_PART) if i != j])


def _mm(a, b):
    return jax.lax.dot(a, b, preferred_element_type=jnp.float32)


def _gelu(v):
    return 0.5 * v * (1.0 + jax.lax.erf(v * np.float32(0.7071067811865476)))


def _mm4(hw, w128):
    return jnp.concatenate(
        [_mm(hw[:, q * 128:(q + 1) * 128], w128) for q in range(4)], axis=1
    )


def _fwd_kernel(
    x_ref, nin_ref,
    node_Wt, node_b,
    ee_Wexp, ee_b1w, ee_W2p, ee_b2w,
    A128, NBt, NCt, eub1w, euW2p, eub2w, e2vp,
    nuAt, nuBt, nub1, nuW2t, nub2,
    offdiag,
    W0he3, W0hv3, w_rp, w_r2, fhb0, fhW1t, fhb1, fhW2t, fhb2,
    out_ref,
):
    Bb = x_ref.shape[0]
    R = Bb * N_PART
    xb = x_ref[...]
    x_r = xb.reshape(R, DIM)

    nin = nin_ref[...].reshape(R, DIM + 1)
    hv = _mm(nin, node_Wt[...]) + node_b[...]

    feats = []
    r2 = None
    for d in range(DIM):
        xjd = jax.lax.broadcast_in_dim(x_r[:, d:d + 1], (R, N_PART), (0, 1))
        xid = jax.lax.broadcast_in_dim(
            xb[:, :, d], (Bb, N_PART, N_PART), (0, 2)
        ).reshape(R, N_PART)
        feats.append(xjd - xid)
        dd = feats[-1] * feats[-1]
        r2 = dd if r2 is None else r2 + dd
    rr = jnp.sqrt(r2 + 1e-12)
    feats.append(rr)
    feats.append(r2)
    ef = jnp.concatenate(feats, axis=1)
    y1 = _gelu(_mm(ef, ee_Wexp[...]) + ee_b1w[...])
    he = _mm4(y1, ee_W2p[...]) + ee_b2w[...]

    for s in range(N_STEPS):
        nb = _mm(hv, NBt[s])
        nc = _mm(hv, NCt[s])
        nb3 = nb.reshape(Bb, N_PART, H)
        nb_w = jnp.concatenate(
            [nb3[:, i, :] for i in range(N_PART)], axis=1
        )
        nc_w = jnp.concatenate([nc] * N_PART, axis=1)
        y1 = (
            _mm4(he, A128[s]).reshape(Bb, N_PART, W_E)
            + nb_w[:, None, :]
            + nc_w.reshape(Bb, N_PART, W_E)
            + eub1w[s]
        )
        heg = _gelu(y1.reshape(R, W_E))
        he = _mm4(heg, euW2p[s]) + eub2w[s]
        msg = _mm4(he, e2vp[s])
        msgm = (msg.reshape(Bb, N_PART, W_E) * offdiag[...][None]).reshape(R, W_E)
        agg = msgm[:, :H]
        for i in range(1, N_PART):
            agg = agg + msgm[:, i * H:(i + 1) * H]
        agg = agg * np.float32(1.0 / (N_PART - 1))
        z = _mm(hv, nuAt[s]) + _mm(agg, nuBt[s]) + nub1[s]
        hv = _mm(_gelu(z), nuW2t[s]) + nub2[s]

    he3 = he.reshape(Bb, N_PART, W_E)
    hv3 = hv.reshape(Bb, N_PART, H)
    acc = jnp.zeros((Bb, H), jnp.float32)
    for j in range(N_PART):
        acc = acc + _mm(he3[:, j, :], W0he3[j])
    for p in range(N_PART):
        acc = acc + _mm(hv3[:, p, :], W0hv3[p])
    r2a = (xb * xb).sum(axis=(1, 2))[:, None]
    d01 = xb[:, 0, :] - xb[:, 1, :]
    rp = jnp.sqrt((d01 * d01).sum(axis=1)[:, None] + 1e-12)
    h0 = _gelu(acc + rp * w_rp[...] + r2a * w_r2[...] + fhb0[...])
    h1 = _gelu(_mm(h0, fhW1t[...]) + fhb1[...])
    out_ref[...] = _mm(h1, fhW2t[...]) + fhb2[...]


def kernel(x, spin, params):
    B = x.shape[0]
    p = params
    f32 = jnp.float32

    nin = jnp.concatenate([x, spin[..., None].astype(f32)], axis=-1)

    I4 = jnp.eye(4, dtype=f32)
    I16 = jnp.eye(N_PART, dtype=f32)

    def kron4(w):
        return jnp.kron(I4, w)

    def tile16(b):
        return jnp.tile(b, (1, N_PART))

    node_Wt = p["node_W"].T
    node_b = p["node_b"][None]

    ee_W1t = p["ee_W1"].T
    ee_Wexp = (
        ee_W1t[:, None, None, :] * I16[None, :, :, None]
    ).reshape(5 * N_PART, W_E)
    ee_b1w = tile16(p["ee_b1"][None])
    ee_W2p = kron4(p["ee_W2"].T)
    ee_b2w = tile16(p["ee_b2"][None])

    W1t_eu = jnp.transpose(p["eu_W1"], (0, 2, 1))
    At, Bt_, Ct = W1t_eu[:, :H], W1t_eu[:, H:2 * H], W1t_eu[:, 2 * H:]
    v2eT = jnp.transpose(p["v2e_W"], (0, 2, 1))
    NBt = jnp.matmul(v2eT, Bt_)
    NCt = jnp.matmul(v2eT, Ct)
    A128 = jnp.stack([kron4(At[s]) for s in range(N_STEPS)])
    eub1w = jnp.stack([tile16(p["eu_b1"][s][None]) for s in range(N_STEPS)])
    euW2p = jnp.stack([kron4(p["eu_W2"][s].T) for s in range(N_STEPS)])
    eub2w = jnp.stack([tile16(p["eu_b2"][s][None]) for s in range(N_STEPS)])
    e2vp = jnp.stack([kron4(p["e2v_W"][s].T) for s in range(N_STEPS)])

    nuW1t = jnp.transpose(p["nu_W1"], (0, 2, 1))
    nuAt, nuBt = nuW1t[:, :H], nuW1t[:, H:]
    nub1 = p["nu_b1"][:, None]
    nuW2t = jnp.transpose(p["nu_W2"], (0, 2, 1))
    nub2 = p["nu_b2"][:, None]

    offdiag = jnp.repeat(1.0 - I16, H, axis=1)

    W0 = p["fh_W0"]
    W0hv3 = W0[:, :N_PART * H].T.reshape(N_PART, H, H)
    W0he_e = W0[:, N_PART * H:N_PART * H + 240 * H].T.reshape(240, H, H)
    W0he_g = (
        jnp.zeros((N_PART, N_PART, H, H), f32)
        .at[_EDGE_I, _EDGE_J].set(W0he_e)
        .transpose(1, 0, 2, 3)
        .reshape(N_PART, W_E, H)
    )
    w_rp = W0[:, 8192][None]
    w_r2 = W0[:, 8193][None]
    fhb0 = p["fh_b0"][None]
    fhW1t = p["fh_W1"].T
    fhb1 = p["fh_b1"][None]
    fhW2t = p["fh_W2"].T
    fhb2 = p["fh_b2"][None]

    weights = [
        node_Wt, node_b,
        ee_Wexp, ee_b1w, ee_W2p, ee_b2w,
        A128, NBt, NCt, eub1w, euW2p, eub2w, e2vp,
        nuAt, nuBt, nub1, nuW2t, nub2,
        offdiag,
        W0he_g, W0hv3, w_rp, w_r2, fhb0, fhW1t, fhb1, fhW2t, fhb2,
    ]

    def wspec(w):
        nd = w.ndim
        return pl.BlockSpec(w.shape, lambda i, _nd=nd: (0,) * _nd)

    grid = (B // B_BLK,)
    out = pl.pallas_call(
        _fwd_kernel,
        grid=grid,
        in_specs=[
            pl.BlockSpec((B_BLK, N_PART, DIM), lambda i: (i, 0, 0)),
            pl.BlockSpec((B_BLK, N_PART, DIM + 1), lambda i: (i, 0, 0)),
        ] + [wspec(w) for w in weights],
        out_specs=pl.BlockSpec((B_BLK, 1), lambda i: (i, 0)),
        out_shape=jax.ShapeDtypeStruct((B, 1), f32),
        compiler_params=pltpu.CompilerParams(
            dimension_semantics=("arbitrary",),
        ),
    )(x, nin, *weights)
    return out


# speedup vs baseline: 2.6376x; 1.1184x over previous
import numpy as np
import jax
import jax.numpy as jnp
from jax.experimental import pallas as pl
from jax.experimental.pallas import tpu as pltpu

N_PART = 16
DIM = 3
H = 32
N_STEPS = 2
B_BLK = 128
NSLICE = 4
W_E = N_PART * H

_EDGE_I = np.asarray([i for i in range(N_PART) for j in range(N_PART) if i != j])
_EDGE_J = np.asarray([j for i in range(N_PART) for j in range(N_PART) if i != j])


def _mm(a, b):
    return jax.lax.dot(a, b, preferred_element_type=jnp.float32)


def _gelu(v):
    return 0.5 * v * (1.0 + jax.lax.erf(v * np.float32(0.7071067811865476)))


def _fwd_kernel(
    x_ref, nin_ref,
    node_Wt, node_b,
    ee_Wexp, ee_b1w, ee_W2p, ee_b2w,
    A128, NBt, NCt, eub1w, euW2p, eub2w,
    nuAt, ENt, CENt, nub1, nuW2t, nub2,
    W0he3, W0hv3, w_rp, w_r2, fhb0, fhW1t, fhb1, fhW2t, fhb2,
    out_ref,
):
    Bb = x_ref.shape[0]
    R = Bb * N_PART
    xb = x_ref[...]
    x_r = xb.reshape(R, DIM)

    def lanes(w, q):
        return w[:, q * 128:(q + 1) * 128]

    nin = nin_ref[...].reshape(R, DIM + 1)
    hv = _mm(nin, node_Wt[...]) + node_b[...]

    feats = []
    r2 = None
    for d in range(DIM):
        xjd = jax.lax.broadcast_in_dim(x_r[:, d:d + 1], (R, N_PART), (0, 1))
        xid = jax.lax.broadcast_in_dim(
            xb[:, :, d], (Bb, N_PART, N_PART), (0, 2)
        ).reshape(R, N_PART)
        feats.append(xjd - xid)
        dd = feats[-1] * feats[-1]
        r2 = dd if r2 is None else r2 + dd
    rr = jnp.sqrt(r2 + 1e-12)
    feats.append(rr)
    feats.append(r2)
    ef = jnp.concatenate(feats, axis=1)
    ee_Wexp_v = ee_Wexp[...]
    ee_W2p_v = ee_W2p[...]
    he = [
        _mm(_gelu(_mm(ef, lanes(ee_Wexp_v, q)) + lanes(ee_b1w[...], q)),
            ee_W2p_v) + lanes(ee_b2w[...], q)
        for q in range(NSLICE)
    ]

    for s in range(N_STEPS):
        nb = _mm(hv, NBt[s])
        nc = _mm(hv, NCt[s])
        nb3 = nb.reshape(Bb, N_PART, H)
        nc4 = jnp.concatenate([nc] * NSLICE, axis=1)
        A_v = A128[s]
        W2_v = euW2p[s]
        zagg = None
        he_new = []
        diag = []
        for q in range(NSLICE):
            nb_wq = jnp.concatenate(
                [nb3[:, NSLICE * q + t, :] for t in range(NSLICE)], axis=1
            )
            y1 = (
                (_mm(he[q], A_v) + nc4 + lanes(eub1w[s], q)).reshape(Bb, N_PART, 128)
                + nb_wq[:, None, :]
            ).reshape(R, 128)
            hq = _mm(_gelu(y1), W2_v) + lanes(eub2w[s], q)
            he_new.append(hq)
            zq = _mm(hq, CENt[s])
            zagg = zq if zagg is None else zagg + zq
            hq3 = hq.reshape(Bb, N_PART, 128)
            diag.extend(
                hq3[:, NSLICE * q + t, t * H:(t + 1) * H][:, None, :]
                for t in range(NSLICE)
            )
        he = he_new
        dg = jnp.concatenate(diag, axis=1).reshape(R, H)
        z = _mm(hv, nuAt[s]) + zagg - _mm(dg, ENt[s]) + nub1[s]
        hv = _mm(_gelu(z), nuW2t[s]) + nub2[s]

    hv3 = hv.reshape(Bb, N_PART, H)
    W0he_v = W0he3[...]
    W0hv_v = W0hv3[...]
    acc = None
    for q in range(NSLICE):
        hq3 = he[q].reshape(Bb, N_PART, 128)
        for j in range(N_PART):
            a = _mm(hq3[:, j, :], W0he_v[q, j])
            acc = a if acc is None else acc + a
    for pnode in range(N_PART):
        acc = acc + _mm(hv3[:, pnode, :], W0hv_v[pnode])
    r2a = (xb * xb).sum(axis=(1, 2))[:, None]
    d01 = xb[:, 0, :] - xb[:, 1, :]
    rp = jnp.sqrt((d01 * d01).sum(axis=1)[:, None] + 1e-12)
    h0 = _gelu(acc + rp * w_rp[...] + r2a * w_r2[...] + fhb0[...])
    h1 = _gelu(_mm(h0, fhW1t[...]) + fhb1[...])
    out_ref[...] = _mm(h1, fhW2t[...]) + fhb2[...]


def kernel(x, spin, params):
    B = x.shape[0]
    p = params
    f32 = jnp.float32

    nin = jnp.concatenate([x, spin[..., None].astype(f32)], axis=-1)

    I4 = jnp.eye(NSLICE, dtype=f32)
    I16 = jnp.eye(N_PART, dtype=f32)

    def kron4(w):
        return jnp.kron(I4, w)

    def tile16(b):
        return jnp.tile(b, (1, N_PART))

    node_Wt = p["node_W"].T
    node_b = p["node_b"][None]

    ee_W1t = p["ee_W1"].T
    ee_Wexp = (
        ee_W1t[:, None, None, :] * I16[None, :, :, None]
    ).reshape(5 * N_PART, W_E)
    ee_b1w = tile16(p["ee_b1"][None])
    ee_W2p = kron4(p["ee_W2"].T)
    ee_b2w = tile16(p["ee_b2"][None])

    W1t_eu = jnp.transpose(p["eu_W1"], (0, 2, 1))
    At, Bt_, Ct = W1t_eu[:, :H], W1t_eu[:, H:2 * H], W1t_eu[:, 2 * H:]
    v2eT = jnp.transpose(p["v2e_W"], (0, 2, 1))
    NBt = jnp.matmul(v2eT, Bt_)
    NCt = jnp.matmul(v2eT, Ct)
    A128 = jnp.stack([kron4(At[s]) for s in range(N_STEPS)])
    eub1w = jnp.stack([tile16(p["eu_b1"][s][None]) for s in range(N_STEPS)])
    euW2p = jnp.stack([kron4(p["eu_W2"][s].T) for s in range(N_STEPS)])
    eub2w = jnp.stack([tile16(p["eu_b2"][s][None]) for s in range(N_STEPS)])

    nuW1t = jnp.transpose(p["nu_W1"], (0, 2, 1))
    nuAt, nuBt = nuW1t[:, :H], nuW1t[:, H:]
    e2vT = jnp.transpose(p["e2v_W"], (0, 2, 1))
    ENt = jnp.matmul(e2vT, nuBt) * (1.0 / (N_PART - 1))
    CENt = jnp.tile(ENt, (1, NSLICE, 1))
    nub1 = p["nu_b1"][:, None]
    nuW2t = jnp.transpose(p["nu_W2"], (0, 2, 1))
    nub2 = p["nu_b2"][:, None]

    W0 = p["fh_W0"]
    W0hv3 = W0[:, :N_PART * H].T.reshape(N_PART, H, H)
    W0he_e = W0[:, N_PART * H:N_PART * H + 240 * H].T.reshape(240, H, H)
    W0he3 = (
        jnp.zeros((N_PART, N_PART, H, H), f32)
        .at[_EDGE_I, _EDGE_J].set(W0he_e)
        .transpose(1, 0, 2, 3)
        .reshape(N_PART, NSLICE, 128, H)
        .transpose(1, 0, 2, 3)
    )
    w_rp = W0[:, 8192][None]
    w_r2 = W0[:, 8193][None]
    fhb0 = p["fh_b0"][None]
    fhW1t = p["fh_W1"].T
    fhb1 = p["fh_b1"][None]
    fhW2t = p["fh_W2"].T
    fhb2 = p["fh_b2"][None]

    weights = [
        node_Wt, node_b,
        ee_Wexp, ee_b1w, ee_W2p, ee_b2w,
        A128, NBt, NCt, eub1w, euW2p, eub2w,
        nuAt, ENt, CENt, nub1, nuW2t, nub2,
        W0he3, W0hv3, w_rp, w_r2, fhb0, fhW1t, fhb1, fhW2t, fhb2,
    ]

    def wspec(w):
        nd = w.ndim
        return pl.BlockSpec(w.shape, lambda i, _nd=nd: (0,) * _nd)

    grid = (B // B_BLK,)
    out = pl.pallas_call(
        _fwd_kernel,
        grid=grid,
        in_specs=[
            pl.BlockSpec((B_BLK, N_PART, DIM), lambda i: (i, 0, 0)),
            pl.BlockSpec((B_BLK, N_PART, DIM + 1), lambda i: (i, 0, 0)),
        ] + [wspec(w) for w in weights],
        out_specs=pl.BlockSpec((B_BLK, 1), lambda i: (i, 0)),
        out_shape=jax.ShapeDtypeStruct((B, 1), f32),
        compiler_params=pltpu.CompilerParams(
            dimension_semantics=("arbitrary",),
        ),
    )(x, nin, *weights)
    return out


# speedup vs baseline: 3.1268x; 1.1855x over previous
import numpy as np
import jax
import jax.numpy as jnp
from jax.experimental import pallas as pl
from jax.experimental.pallas import tpu as pltpu

N_PART = 16
DIM = 3
H = 32
N_STEPS = 2
B_BLK = 128
NSLICE = 4
W_E = N_PART * H

_EDGE_I = np.asarray([i for i in range(N_PART) for j in range(N_PART) if i != j])
_EDGE_J = np.asarray([j for i in range(N_PART) for j in range(N_PART) if i != j])


def _mm(a, b):
    return jax.lax.dot(a, b, preferred_element_type=jnp.float32)


def _mmb(a, b):
    return jax.lax.dot(
        a.astype(jnp.bfloat16), b.astype(jnp.bfloat16),
        preferred_element_type=jnp.float32,
    )


def _gelu(v):
    return 0.5 * v * (1.0 + jax.lax.erf(v * np.float32(0.7071067811865476)))


def _gelu2(v):
    return v * (1.0 + jax.lax.erf(v * np.float32(0.7071067811865476)))


def _fwd_kernel(
    x_ref, xt_ref, nint_ref,
    node_Wt, node_b,
    ee_Wexp, ee_b1w, ee_W2p, ee_b2w,
    A128, NBt, NCt, eub1, euW2p, eub2w,
    nuAt, ENt, CENt, nub1, nuW2t, nub2,
    W0he3, W0hv3, w_rp, w_r2, fhb0, fhW1t, fhb1, fhW2t, fhb2,
    out_ref,
):
    Bb = x_ref.shape[0]
    R = Bb * N_PART
    xb = x_ref[...]
    xt = xt_ref[...]
    x_r = xt.reshape(R, DIM)

    def lanes(w, q):
        return w[:, q * 128:(q + 1) * 128]

    nin = nint_ref[...].reshape(R, DIM + 1)
    hv = _mm(nin, node_Wt[...]) + node_b[...]

    feats = []
    r2 = None
    for d in range(DIM):
        xjd = jax.lax.broadcast_in_dim(x_r[:, d:d + 1], (R, N_PART), (0, 1))
        xid = jax.lax.broadcast_in_dim(
            xb[:, :, d], (N_PART, Bb, N_PART), (1, 2)
        ).reshape(R, N_PART)
        feats.append(xjd - xid)
        dd = feats[-1] * feats[-1]
        r2 = dd if r2 is None else r2 + dd
    rr = jnp.sqrt(r2 + 1e-12)
    feats.append(rr)
    feats.append(r2)
    ef = jnp.concatenate(feats, axis=1)
    ee_Wexp_v = ee_Wexp[...]
    ee_W2p_v = ee_W2p[...]
    he = [
        _mmb(_gelu2(_mmb(ef, lanes(ee_Wexp_v, q)) + lanes(ee_b1w[...], q)),
             ee_W2p_v) + lanes(ee_b2w[...], q)
        for q in range(NSLICE)
    ]

    for s in range(N_STEPS):
        nb = _mm(hv, NBt[s])
        nc = _mm(hv, NCt[s]) + eub1[s]
        nb3 = nb.reshape(N_PART, Bb, H)
        nc4 = jnp.concatenate([nc] * NSLICE, axis=1)
        A_v = A128[s]
        W2_v = euW2p[s]
        zagg = None
        he_new = []
        diag = []
        for q in range(NSLICE):
            nb_wq = jnp.concatenate(
                [nb3[NSLICE * q + t] for t in range(NSLICE)], axis=1
            )
            y1 = (
                (_mmb(he[q], A_v) + nc4).reshape(N_PART, Bb, 128)
                + nb_wq[None]
            ).reshape(R, 128)
            hq = _mmb(_gelu2(y1), W2_v) + lanes(eub2w[s], q)
            he_new.append(hq)
            zq = _mmb(hq, CENt[s])
            zagg = zq if zagg is None else zagg + zq
            hq3 = hq.reshape(N_PART, Bb, 128)
            diag.extend(
                hq3[NSLICE * q + t, :, t * H:(t + 1) * H]
                for t in range(NSLICE)
            )
        he = he_new
        dg = jnp.concatenate(diag, axis=0)
        z = _mm(hv, nuAt[s]) + zagg - _mm(dg, ENt[s]) + nub1[s]
        hv = _mm(_gelu2(z), nuW2t[s]) + nub2[s]

    hv3 = hv.reshape(N_PART, Bb, H)
    W0he_v = W0he3[...]
    W0hv_v = W0hv3[...]
    acc = None
    for q in range(NSLICE):
        hq3 = he[q].reshape(N_PART, Bb, 128)
        for j in range(N_PART):
            a = _mmb(hq3[j], W0he_v[q, j])
            acc = a if acc is None else acc + a
    for pnode in range(N_PART):
        acc = acc + _mm(hv3[pnode], W0hv_v[pnode])
    r2a = (xb * xb).sum(axis=(1, 2))[:, None]
    d01 = xb[:, 0, :] - xb[:, 1, :]
    rp = jnp.sqrt((d01 * d01).sum(axis=1)[:, None] + 1e-12)
    h0 = _gelu2(acc + rp * w_rp[...] + r2a * w_r2[...] + fhb0[...])
    h1 = _gelu2(_mm(h0, fhW1t[...]) + fhb1[...])
    out_ref[...] = _mm(h1, fhW2t[...]) + fhb2[...]


def kernel(x, spin, params):
    B = x.shape[0]
    p = params
    f32 = jnp.float32

    nin = jnp.concatenate([x, spin[..., None].astype(f32)], axis=-1)
    nin_t = jnp.transpose(nin, (1, 0, 2))
    x_t = jnp.transpose(x, (1, 0, 2))

    I4 = jnp.eye(NSLICE, dtype=f32)
    I16 = jnp.eye(N_PART, dtype=f32)

    def kron4(w):
        return jnp.kron(I4, w)

    def tile16(b):
        return jnp.tile(b, (1, N_PART))

    node_Wt = p["node_W"].T
    node_b = p["node_b"][None]

    ee_W1t = p["ee_W1"].T
    ee_Wexp = (
        ee_W1t[:, None, None, :] * I16[None, :, :, None]
    ).reshape(5 * N_PART, W_E)
    ee_b1w = tile16(p["ee_b1"][None])
    ee_W2p = kron4(0.5 * p["ee_W2"].T)
    ee_b2w = tile16(p["ee_b2"][None])

    W1t_eu = jnp.transpose(p["eu_W1"], (0, 2, 1))
    At, Bt_, Ct = W1t_eu[:, :H], W1t_eu[:, H:2 * H], W1t_eu[:, 2 * H:]
    v2eT = jnp.transpose(p["v2e_W"], (0, 2, 1))
    NBt = jnp.matmul(v2eT, Bt_)
    NCt = jnp.matmul(v2eT, Ct)
    A128 = jnp.stack([kron4(At[s]) for s in range(N_STEPS)])
    eub1 = p["eu_b1"][:, None]
    euW2p = jnp.stack([kron4(0.5 * p["eu_W2"][s].T) for s in range(N_STEPS)])
    eub2w = jnp.stack([tile16(p["eu_b2"][s][None]) for s in range(N_STEPS)])

    nuW1t = jnp.transpose(p["nu_W1"], (0, 2, 1))
    nuAt, nuBt = nuW1t[:, :H], nuW1t[:, H:]
    e2vT = jnp.transpose(p["e2v_W"], (0, 2, 1))
    ENt = jnp.matmul(e2vT, nuBt) * (1.0 / (N_PART - 1))
    CENt = jnp.tile(ENt, (1, NSLICE, 1))
    nub1 = p["nu_b1"][:, None]
    nuW2t = 0.5 * jnp.transpose(p["nu_W2"], (0, 2, 1))
    nub2 = p["nu_b2"][:, None]

    W0 = p["fh_W0"]
    W0hv3 = W0[:, :N_PART * H].T.reshape(N_PART, H, H)
    W0he_e = W0[:, N_PART * H:N_PART * H + 240 * H].T.reshape(240, H, H)
    W0he3 = (
        jnp.zeros((N_PART, N_PART, H, H), f32)
        .at[_EDGE_I, _EDGE_J].set(W0he_e)
        .transpose(1, 0, 2, 3)
        .reshape(N_PART, NSLICE, 128, H)
        .transpose(1, 0, 2, 3)
    )
    w_rp = W0[:, 8192][None]
    w_r2 = W0[:, 8193][None]
    fhb0 = p["fh_b0"][None]
    fhW1t = 0.5 * p["fh_W1"].T
    fhb1 = p["fh_b1"][None]
    fhW2t = 0.5 * p["fh_W2"].T
    fhb2 = p["fh_b2"][None]

    weights = [
        node_Wt, node_b,
        ee_Wexp, ee_b1w, ee_W2p, ee_b2w,
        A128, NBt, NCt, eub1, euW2p, eub2w,
        nuAt, ENt, CENt, nub1, nuW2t, nub2,
        W0he3, W0hv3, w_rp, w_r2, fhb0, fhW1t, fhb1, fhW2t, fhb2,
    ]

    def wspec(w):
        nd = w.ndim
        return pl.BlockSpec(w.shape, lambda i, _nd=nd: (0,) * _nd)

    grid = (B // B_BLK,)
    out = pl.pallas_call(
        _fwd_kernel,
        grid=grid,
        in_specs=[
            pl.BlockSpec((B_BLK, N_PART, DIM), lambda i: (i, 0, 0)),
            pl.BlockSpec((N_PART, B_BLK, DIM), lambda i: (0, i, 0)),
            pl.BlockSpec((N_PART, B_BLK, DIM + 1), lambda i: (0, i, 0)),
        ] + [wspec(w) for w in weights],
        out_specs=pl.BlockSpec((B_BLK, 1), lambda i: (i, 0)),
        out_shape=jax.ShapeDtypeStruct((B, 1), f32),
        compiler_params=pltpu.CompilerParams(
            dimension_semantics=("arbitrary",),
        ),
    )(x, x_t, nin_t, *weights)
    return out
